# baseline (speedup 1.0000x reference)
import functools

import jax
import jax.numpy as jnp
from jax.experimental import pallas as pl
from jax.experimental.pallas import tpu as pltpu

N = 10000
E = 160000
G = 64
K_SIGN = 1000.0
EPS_SIGN = 5.0
HID = 128
OUT = 128

_RB = 256
_CB = 2048
_P = 10240


def _sign_reduce_body(ccol_ref, crow_ref, out_ref):
    j = pl.program_id(1)
    t = jnp.tanh(K_SIGN * (ccol_ref[...] - crow_ref[...]) - EPS_SIGN)
    colid = j * _CB + jax.lax.broadcasted_iota(jnp.int32, (_RB, _CB), 1)
    t = jnp.where(colid < N, t, 0.0)
    part = jnp.sum(t, axis=1, keepdims=True)

    @pl.when(j == 0)
    def _init():
        out_ref[...] = part

    @pl.when(j > 0)
    def _acc():
        out_ref[...] += part


@jax.jit
def _sign_reduce_pallas(c):
    cpad = jnp.zeros((_P,), jnp.float32).at[:N].set(c[:, 0])
    ccol = cpad[:, None]
    crow = cpad[None, :]
    out = pl.pallas_call(
        _sign_reduce_body,
        grid=(_P // _RB, _P // _CB),
        in_specs=[
            pl.BlockSpec((_RB, 1), lambda i, j: (i, 0)),
            pl.BlockSpec((1, _CB), lambda i, j: (0, j)),
        ],
        out_specs=pl.BlockSpec((_RB, 1), lambda i, j: (i, 0)),
        out_shape=jax.ShapeDtypeStruct((_P, 1), jnp.float32),
        compiler_params=pltpu.CompilerParams(
            dimension_semantics=("parallel", "arbitrary"),
        ),
    )(ccol, crow)
    return out[:N]


def _info_collect(h, src, dst, W):
    agg = jax.ops.segment_sum(h[src], dst, num_segments=h.shape[0])
    return jax.nn.relu((h + agg) @ W)


def _seg_softmax(v, batch, num_segments):
    m = jax.ops.segment_max(v, batch, num_segments=num_segments)
    m = jnp.where(jnp.isfinite(m), m, 0.0)
    e = jnp.exp(v - m[batch])
    s = jax.ops.segment_sum(e, batch, num_segments=num_segments)
    return e / (s[batch] + 1e-16)


def kernel(x, edge_index, batch, params):
    src = edge_index[0]
    dst = edge_index[1]
    counts = jnp.bincount(batch, length=G).astype(jnp.float32)
    cum = jnp.concatenate([jnp.zeros((1,), jnp.float32), jnp.cumsum(counts)])
    scale = jnp.maximum(counts[batch][:, None], 1.0)

    c = _info_collect(x, src, dst, params['W_ic1'])
    c = _info_collect(c, src, dst, params['W_ic2'])
    c = _info_collect(c, src, dst, params['W_ic3'])
    D = _sign_reduce_pallas(c)
    yv = jax.nn.sigmoid(params['f5_a'] * D / scale + params['f5_b'])
    agg = jax.ops.segment_sum(yv[src], dst, num_segments=N)
    out = (yv + agg) @ params['W_indi']
    relu_out = jax.nn.relu(out)[:, 0]
    per_idx = jnp.arange(N, dtype=jnp.float32) - cum[batch]
    p = _seg_softmax(relu_out / 0.1, batch, G)
    expected = jax.ops.segment_sum(p * per_idx, batch, num_segments=G)
    node_indices = expected[:, None]
    p2 = _seg_softmax(relu_out / 0.5, batch, G)
    combined = p2[:, None] * relu_out[:, None]
    y = combined + yv

    c = _info_collect(y, src, dst, params['W_ic1'])
    c = _info_collect(c, src, dst, params['W_ic2'])
    c = _info_collect(c, src, dst, params['W_ic3'])
    D = _sign_reduce_pallas(c)
    yv = jax.nn.sigmoid(params['f5_a'] * D / scale + params['f5_b'])
    h = yv
    for l in range(3):
        agg = jax.ops.segment_sum(h[src], dst, num_segments=N)
        h2 = (1.0 + params['gin_eps'][l]) * h + agg
        h2 = jax.nn.relu(h2 @ params['gin_W1_' + str(l)]) @ params['gin_W2_' + str(l)]
        if l < 2:
            h2 = jax.nn.relu(h2)
        h = h2
    res = jax.ops.segment_sum(h, batch, num_segments=G)
    return res, node_indices


# speedup vs baseline: 7.0762x; 7.0762x over previous
import functools

import jax
import jax.numpy as jnp
from jax import lax
from jax.experimental import pallas as pl
from jax.experimental.pallas import tpu as pltpu
from jax.experimental.pallas import tpu_sc as plsc

N = 10000
E = 160000
G = 64
K_SIGN = 1000.0
EPS_SIGN = 5.0
HID = 128
OUT = 128

_NC = 2
_NS = 16
_NW = _NC * _NS
_EW = E // _NW
_NPAD = 10240
_RPT = _NPAD // _NS
_ZCH = 128


_RB = 256
_CB = 2048
_P = 10240


def _sign_reduce_body(ccol_ref, crow_ref, out_ref):
    j = pl.program_id(1)
    t = jnp.tanh(K_SIGN * (ccol_ref[...] - crow_ref[...]) - EPS_SIGN)
    colid = j * _CB + jax.lax.broadcasted_iota(jnp.int32, (_RB, _CB), 1)
    t = jnp.where(colid < N, t, 0.0)
    part = jnp.sum(t, axis=1, keepdims=True)

    @pl.when(j == 0)
    def _init():
        out_ref[...] = part

    @pl.when(j > 0)
    def _acc():
        out_ref[...] += part


@jax.jit
def _sign_reduce_pallas(c):
    cpad = jnp.zeros((_P,), jnp.float32).at[:N].set(c[:, 0])
    out = pl.pallas_call(
        _sign_reduce_body,
        grid=(_P // _RB, _P // _CB),
        in_specs=[
            pl.BlockSpec((_RB, 1), lambda i, j: (i, 0)),
            pl.BlockSpec((1, _CB), lambda i, j: (0, j)),
        ],
        out_specs=pl.BlockSpec((_RB, 1), lambda i, j: (i, 0)),
        out_shape=jax.ShapeDtypeStruct((_P, 1), jnp.float32),
        compiler_params=pltpu.CompilerParams(
            dimension_semantics=("parallel", "arbitrary"),
        ),
    )(cpad[:, None], cpad[None, :])
    return out[:N]


@functools.lru_cache(maxsize=None)
def _make_segsum_sc(d, ch):
    nch = _EW // ch
    mesh = plsc.VectorSubcoreMesh(core_axis_name="c", subcore_axis_name="s")

    @functools.partial(
        pl.kernel,
        mesh=mesh,
        compiler_params=pltpu.CompilerParams(use_tc_tiling_on_sc=False),
        out_type=jax.ShapeDtypeStruct((_NC, _NPAD, d), jnp.float32),
        scratch_types=[
            pltpu.VMEM((ch,), jnp.int32),
            pltpu.VMEM((ch,), jnp.int32),
            pltpu.VMEM((ch, d), jnp.float32),
            pltpu.VMEM((_ZCH, d), jnp.float32),
            pltpu.VMEM_SHARED((_NPAD, d), jnp.float32),
            pltpu.SemaphoreType.DMA,
        ],
    )
    def seg(h_hbm, src_hbm, dst_hbm, zeros_hbm, out_hbm,
            src_ch, dst_ch, stage, zb, acc, sem):
        c = lax.axis_index("c")
        s = lax.axis_index("s")
        wid = c * _NS + s
        pltpu.sync_copy(zeros_hbm, zb)
        for z in range(_RPT // _ZCH):
            pltpu.sync_copy(zb, acc.at[pl.ds(s * _RPT + z * _ZCH, _ZCH)])
        plsc.subcore_barrier()
        for k in range(nch):
            pltpu.sync_copy(src_hbm.at[pl.ds(wid * _EW + k * ch, ch)], src_ch)
            pltpu.sync_copy(dst_hbm.at[pl.ds(wid * _EW + k * ch, ch)], dst_ch)
            pltpu.async_copy(h_hbm.at[src_ch], stage, sem).wait()
            pltpu.sync_copy(stage, acc.at[dst_ch], add=True)
        plsc.subcore_barrier()
        for z in range(_RPT // _ZCH):
            sl = pl.ds(s * _RPT + z * _ZCH, _ZCH)
            pltpu.sync_copy(acc.at[sl], zb)
            pltpu.sync_copy(zb, out_hbm.at[c].at[sl])

    return seg


def _segsum_sc(h, src3d, dst3d, zeros, ch):
    d = h.shape[1]
    parts = _make_segsum_sc(d, ch)(h, src3d, dst3d, zeros)
    return parts[0, :N] + parts[1, :N]


def _seg_softmax(v, batch, num_segments):
    m = jax.ops.segment_max(v, batch, num_segments=num_segments)
    m = jnp.where(jnp.isfinite(m), m, 0.0)
    e = jnp.exp(v - m[batch])
    s = jax.ops.segment_sum(e, batch, num_segments=num_segments)
    return e / (s[batch] + 1e-16)


def kernel(x, edge_index, batch, params):
    src = edge_index[0]
    dst = edge_index[1]
    ch_s, ch_b = 1000, 200
    z8 = jnp.zeros((_ZCH, 8), jnp.float32)
    z128 = jnp.zeros((_ZCH, HID), jnp.float32)

    def seg_small(h):
        d = h.shape[1]
        hp = h if d == 8 else jnp.pad(h, ((0, 0), (0, 8 - d)))
        agg = _segsum_sc(hp, src, dst, z8, ch_s)
        return agg[:, :d]

    def seg_big(h):
        return _segsum_sc(h, src, dst, z128, ch_b)

    def info_collect(h, W):
        return jax.nn.relu((h + seg_small(h)) @ W)

    counts = jnp.bincount(batch, length=G).astype(jnp.float32)
    cum = jnp.concatenate([jnp.zeros((1,), jnp.float32), jnp.cumsum(counts)])
    scale = jnp.maximum(counts[batch][:, None], 1.0)

    c = info_collect(x, params['W_ic1'])
    c = info_collect(c, params['W_ic2'])
    c = info_collect(c, params['W_ic3'])
    D = _sign_reduce_pallas(c)
    yv = jax.nn.sigmoid(params['f5_a'] * D / scale + params['f5_b'])
    agg = seg_small(yv)
    out = (yv + agg) @ params['W_indi']
    relu_out = jax.nn.relu(out)[:, 0]
    per_idx = jnp.arange(N, dtype=jnp.float32) - cum[batch]
    p = _seg_softmax(relu_out / 0.1, batch, G)
    expected = jax.ops.segment_sum(p * per_idx, batch, num_segments=G)
    node_indices = expected[:, None]
    p2 = _seg_softmax(relu_out / 0.5, batch, G)
    y = p2[:, None] * relu_out[:, None] + yv

    c = info_collect(y, params['W_ic1'])
    c = info_collect(c, params['W_ic2'])
    c = info_collect(c, params['W_ic3'])
    D = _sign_reduce_pallas(c)
    yv = jax.nn.sigmoid(params['f5_a'] * D / scale + params['f5_b'])
    h = yv
    for l in range(3):
        agg = seg_small(h) if h.shape[1] <= 8 else seg_big(h)
        h2 = (1.0 + params['gin_eps'][l]) * h + agg
        h2 = jax.nn.relu(h2 @ params['gin_W1_' + str(l)]) @ params['gin_W2_' + str(l)]
        if l < 2:
            h2 = jax.nn.relu(h2)
        h = h2
    res = jax.ops.segment_sum(h, batch, num_segments=G)
    return res, node_indices
